# baseline (speedup 1.0000x reference)
import jax
import jax.numpy as jnp
from jax import lax
from jax.experimental import pallas as pl
from jax.experimental.pallas import tpu as pltpu

_C = 1000
_P = 16
_D = 512
_ROWS = _C * _P

_TC_BLK = 100
_TC_NCHUNK = _C // _TC_BLK


def _tc_copy_body(in_any, out_any, idx_ref, buf, rsem, wsem):
    def rd(k):
        return pltpu.make_async_copy(in_any.at[pl.ds(k * _TC_BLK, _TC_BLK)],
                                     buf.at[k], rsem.at[k])

    def wr(k):
        return pltpu.make_async_copy(buf.at[k],
                                     out_any.at[pl.ds(k * _TC_BLK, _TC_BLK)],
                                     wsem.at[k])

    for k in range(_TC_NCHUNK):
        rd(k).start()
    i = lax.broadcasted_iota(jnp.int32, (125, 128), 0)
    j = lax.broadcasted_iota(jnp.int32, (125, 128), 1)
    idx_ref[...] = (i * 128 + j) >> 4
    for k in range(_TC_NCHUNK):
        rd(k).wait()
        wr(k).start()
    for k in range(_TC_NCHUNK):
        wr(k).wait()


def kernel(vision_protos):
    stacked, idx2d = pl.pallas_call(
        _tc_copy_body,
        in_specs=[pl.BlockSpec(memory_space=pl.ANY)],
        out_specs=[pl.BlockSpec(memory_space=pl.ANY),
                   pl.BlockSpec((125, 128), lambda: (0, 0))],
        out_shape=[jax.ShapeDtypeStruct((_C, _P, _D), jnp.float32),
                   jax.ShapeDtypeStruct((125, 128), jnp.int32)],
        scratch_shapes=[
            pltpu.VMEM((_TC_NCHUNK, _TC_BLK, _P, _D), jnp.float32),
            pltpu.SemaphoreType.DMA((_TC_NCHUNK,)),
            pltpu.SemaphoreType.DMA((_TC_NCHUNK,)),
        ],
    )(vision_protos)

    return (stacked.reshape(_ROWS, _D), idx2d.reshape(_ROWS))


# speedup vs baseline: 1.0068x; 1.0068x over previous
import jax
import jax.numpy as jnp
from jax import lax
from jax.experimental import pallas as pl
from jax.experimental.pallas import tpu as pltpu

_C = 1000
_P = 16
_D = 512
_ROWS = _C * _P

_TC_BLK = 125
_TC_NCHUNK = _C // _TC_BLK


def _tc_copy_body(in_any, out_any, idx_ref, buf, rsem, wsem):
    def rd(k):
        return pltpu.make_async_copy(in_any.at[pl.ds(k * _TC_BLK, _TC_BLK)],
                                     buf.at[k], rsem.at[k])

    def wr(k):
        return pltpu.make_async_copy(buf.at[k],
                                     out_any.at[pl.ds(k * _TC_BLK, _TC_BLK)],
                                     wsem.at[k])

    for k in range(_TC_NCHUNK):
        rd(k).start()
    i = lax.broadcasted_iota(jnp.int32, (125, 128), 0)
    j = lax.broadcasted_iota(jnp.int32, (125, 128), 1)
    idx_ref[...] = (i * 128 + j) >> 4
    for k in range(_TC_NCHUNK):
        rd(k).wait()
        wr(k).start()
    for k in range(_TC_NCHUNK):
        wr(k).wait()


def kernel(vision_protos):
    stacked, idx2d = pl.pallas_call(
        _tc_copy_body,
        in_specs=[pl.BlockSpec(memory_space=pl.ANY)],
        out_specs=[pl.BlockSpec(memory_space=pl.ANY),
                   pl.BlockSpec((125, 128), lambda: (0, 0))],
        out_shape=[jax.ShapeDtypeStruct((_C, _P, _D), jnp.float32),
                   jax.ShapeDtypeStruct((125, 128), jnp.int32)],
        scratch_shapes=[
            pltpu.VMEM((_TC_NCHUNK, _TC_BLK, _P, _D), jnp.float32),
            pltpu.SemaphoreType.DMA((_TC_NCHUNK,)),
            pltpu.SemaphoreType.DMA((_TC_NCHUNK,)),
        ],
    )(vision_protos)

    return (stacked.reshape(_ROWS, _D), idx2d.reshape(_ROWS))
